# baseline (speedup 1.0000x reference)
import functools

import jax
import jax.numpy as jnp
from jax import lax
from jax.experimental import pallas as pl
from jax.experimental.pallas import tpu as pltpu
from jax.experimental.pallas import tpu_sc as plsc

_VOCAB = 100000
_DIM = 128
_CTX = 20
_LATENT = 128
_BV = 6272
_NB = (_VOCAB + _BV - 1) // _BV
_S = 8
_NBH = _NB // _S
_VP = _NB * _BV


def _gather_sc(idx, table):
    mesh = plsc.ScalarSubcoreMesh(axis_name="c", num_cores=1)

    @functools.partial(
        pl.kernel,
        mesh=mesh,
        out_type=jax.ShapeDtypeStruct((_CTX, _DIM), jnp.float32),
        scratch_types=[
            pltpu.SMEM((_CTX,), jnp.int32),
            pltpu.SemaphoreType.DMA,
        ],
    )
    def k(idx_hbm, table_hbm, out_hbm, idx_s, sem):
        pltpu.sync_copy(idx_hbm, idx_s)
        copies = []
        for c in range(_CTX):
            copies.append(pltpu.async_copy(
                table_hbm.at[pl.ds(idx_s[c], 1)],
                out_hbm.at[pl.ds(c, 1)], sem))
        for cp in copies:
            cp.wait()

    return k(idx, table)


def _tc_main(embeds, W1, b1r, W2, b2r):
    def body(*refs):
        (emb_ref, w1_ref, b1_ref), w2_refs, b2_refs = (
            refs[:3], refs[3:3 + 2 * _S:2], refs[4:3 + 2 * _S:2])
        out_ref = refs[3 + 2 * _S]
        scr_ref, h_ref, m_ref, s_ref = refs[4 + 2 * _S:]
        i = pl.program_id(0)

        @pl.when(i == 0)
        def _init():
            acc = jnp.zeros((1, _LATENT), jnp.float32)
            for c in range(_CTX):
                e_c = emb_ref[pl.ds(c, 1), :]
                w1s = w1_ref[:, pl.ds(c * _DIM, _DIM)]
                acc = acc + lax.dot_general(
                    e_c, w1s, (((1,), (1,)), ((), ())),
                    preferred_element_type=jnp.float32)
            h_ref[...] = jnp.maximum(acc + b1_ref[...], 0.0)
            m_ref[...] = jnp.full((1, 1), -jnp.inf, jnp.float32)
            s_ref[...] = jnp.zeros((1, 1), jnp.float32)

        h = h_ref[...]
        col = lax.broadcasted_iota(jnp.int32, (1, _BV), 1)
        ls = []
        for s in range(_S):
            blk = s * _NBH + i
            l = (lax.dot_general(
                h, w2_refs[s][...], (((1,), (1,)), ((), ())),
                preferred_element_type=jnp.float32,
                precision=lax.Precision.DEFAULT)
                 + b2_refs[s][...])
            if s == _S - 1:
                l = jnp.where((blk * _BV + col) < _VOCAB, l, -jnp.inf)
            scr_ref[s * _NBH + i] = l
            ls.append(l)
        m_old = m_ref[...]
        s_old = s_ref[...]
        bm = m_old
        for l in ls:
            bm = jnp.maximum(bm, jnp.max(l, axis=1, keepdims=True))
        m_new = bm
        se = jnp.zeros((1, 1), jnp.float32)
        for l in ls:
            se = se + jnp.sum(jnp.exp(l - m_new), axis=1, keepdims=True)
        s_new = s_old * jnp.exp(m_old - m_new) + se
        m_ref[...] = m_new
        s_ref[...] = s_new

        @pl.when(i == _NBH - 1)
        def _fin():
            lse = m_new + jnp.log(s_new)
            for j in range(_NB):
                w = min(_BV, _VOCAB - j * _BV)
                out_ref[:, pl.ds(j * _BV, w)] = scr_ref[j][:, :w] - lse

    w2_specs = []
    for s in range(_S):
        w2_specs.append(pl.BlockSpec(
            (_BV, _DIM), functools.partial(
                lambda s, i: (s * _NBH + i, 0), s)))
        w2_specs.append(pl.BlockSpec(
            (1, _BV), functools.partial(
                lambda s, i: (0, s * _NBH + i), s)))
    return pl.pallas_call(
        body,
        grid=(_NBH,),
        in_specs=[
            pl.BlockSpec((_CTX, _DIM), lambda i: (0, 0)),
            pl.BlockSpec((_LATENT, _CTX * _DIM), lambda i: (0, 0)),
            pl.BlockSpec((1, _LATENT), lambda i: (0, 0)),
        ] + w2_specs,
        out_specs=pl.BlockSpec((1, _VOCAB), lambda i: (0, 0)),
        out_shape=jax.ShapeDtypeStruct((1, _VOCAB), jnp.float32),
        scratch_shapes=[
            pltpu.VMEM((_NB, 1, _BV), jnp.float32),
            pltpu.VMEM((1, _LATENT), jnp.float32),
            pltpu.VMEM((1, 1), jnp.float32),
            pltpu.VMEM((1, 1), jnp.float32),
        ],
        compiler_params=pltpu.CompilerParams(
            dimension_semantics=("arbitrary",)),
    )(embeds, W1, b1r, *([W2, b2r] * _S))


def kernel(inputs, table, W1, b1, W2, b2):
    idx = inputs.astype(jnp.int32)
    embeds = _gather_sc(idx, table)
    return _tc_main(embeds, W1, b1.reshape(1, _LATENT), W2,
                    b2.reshape(1, _VOCAB))


# speedup vs baseline: 1.0987x; 1.0987x over previous
import functools

import jax
import jax.numpy as jnp
from jax import lax
from jax.experimental import pallas as pl
from jax.experimental.pallas import tpu as pltpu
from jax.experimental.pallas import tpu_sc as plsc

_VOCAB = 100000
_DIM = 128
_CTX = 20
_LATENT = 128
_BV = 5120
_NB = (_VOCAB + _BV - 1) // _BV
_S = 4
_NBH = _NB // _S
_VP = _NB * _BV


def _gather_sc(idx, table):
    mesh = plsc.ScalarSubcoreMesh(axis_name="c", num_cores=1)

    @functools.partial(
        pl.kernel,
        mesh=mesh,
        out_type=jax.ShapeDtypeStruct((_CTX, _DIM), jnp.float32),
        scratch_types=[
            pltpu.SMEM((_CTX,), jnp.int32),
            pltpu.SemaphoreType.DMA,
        ],
    )
    def k(idx_hbm, table_hbm, out_hbm, idx_s, sem):
        pltpu.sync_copy(idx_hbm, idx_s)
        copies = []
        for c in range(_CTX):
            copies.append(pltpu.async_copy(
                table_hbm.at[pl.ds(idx_s[c], 1)],
                out_hbm.at[pl.ds(c, 1)], sem))
        for cp in copies:
            cp.wait()

    return k(idx, table)


def _tc_main(embeds, W1, b1r, W2, b2r):
    def body(*refs):
        (emb_ref, w1_ref, b1_ref), w2_refs, b2_refs = (
            refs[:3], refs[3:3 + 2 * _S:2], refs[4:3 + 2 * _S:2])
        out_ref = refs[3 + 2 * _S]
        scr_ref, h_ref, m_ref, s_ref = refs[4 + 2 * _S:]
        i = pl.program_id(0)

        @pl.when(i == 0)
        def _init():
            acc = jnp.zeros((1, _LATENT), jnp.float32)
            for c in range(_CTX):
                e_c = emb_ref[pl.ds(c, 1), :]
                w1s = w1_ref[:, pl.ds(c * _DIM, _DIM)]
                acc = acc + lax.dot_general(
                    e_c, w1s, (((1,), (1,)), ((), ())),
                    preferred_element_type=jnp.float32)
            h_ref[...] = jnp.maximum(acc + b1_ref[...], 0.0)
            m_ref[...] = jnp.full((1, 1), -jnp.inf, jnp.float32)
            s_ref[...] = jnp.zeros((1, 1), jnp.float32)

        h = h_ref[...]
        col = lax.broadcasted_iota(jnp.int32, (1, _BV), 1)
        ls = []
        for s in range(_S):
            blk = s * _NBH + i
            l = (lax.dot_general(
                h, w2_refs[s][...], (((1,), (1,)), ((), ())),
                preferred_element_type=jnp.float32,
                precision=lax.Precision.DEFAULT)
                 + b2_refs[s][...])
            if s == _S - 1:
                l = jnp.where((blk * _BV + col) < _VOCAB, l, -jnp.inf)
            scr_ref[s * _NBH + i] = l
            ls.append(l)
        m_old = m_ref[...]
        s_old = s_ref[...]
        bm = m_old
        for l in ls:
            bm = jnp.maximum(bm, jnp.max(l, axis=1, keepdims=True))
        m_new = bm
        se = jnp.zeros((1, 1), jnp.float32)
        for l in ls:
            se = se + jnp.sum(jnp.exp(l - m_new), axis=1, keepdims=True)
        s_new = s_old * jnp.exp(m_old - m_new) + se
        m_ref[...] = m_new
        s_ref[...] = s_new

        @pl.when(i == _NBH - 1)
        def _fin():
            lse = m_new + jnp.log(s_new)
            for j in range(_NB):
                w = min(_BV, _VOCAB - j * _BV)
                out_ref[:, pl.ds(j * _BV, w)] = scr_ref[j][:, :w] - lse

    w2_specs = []
    for s in range(_S):
        w2_specs.append(pl.BlockSpec(
            (_BV, _DIM), functools.partial(
                lambda s, i: (s * _NBH + i, 0), s)))
        w2_specs.append(pl.BlockSpec(
            (1, _BV), functools.partial(
                lambda s, i: (0, s * _NBH + i), s)))
    return pl.pallas_call(
        body,
        grid=(_NBH,),
        in_specs=[
            pl.BlockSpec((_CTX, _DIM), lambda i: (0, 0)),
            pl.BlockSpec((_LATENT, _CTX * _DIM), lambda i: (0, 0)),
            pl.BlockSpec((1, _LATENT), lambda i: (0, 0)),
        ] + w2_specs,
        out_specs=pl.BlockSpec((1, _VOCAB), lambda i: (0, 0)),
        out_shape=jax.ShapeDtypeStruct((1, _VOCAB), jnp.float32),
        scratch_shapes=[
            pltpu.VMEM((_NB, 1, _BV), jnp.float32),
            pltpu.VMEM((1, _LATENT), jnp.float32),
            pltpu.VMEM((1, 1), jnp.float32),
            pltpu.VMEM((1, 1), jnp.float32),
        ],
        compiler_params=pltpu.CompilerParams(
            dimension_semantics=("arbitrary",)),
    )(embeds, W1, b1r, *([W2, b2r] * _S))


def kernel(inputs, table, W1, b1, W2, b2):
    idx = inputs.astype(jnp.int32)
    embeds = _gather_sc(idx, table)
    return _tc_main(embeds, W1, b1.reshape(1, _LATENT), W2,
                    b2.reshape(1, _VOCAB))
